# baseline (speedup 1.0000x reference)
import functools

import jax
import jax.numpy as jnp
from jax import lax
from jax.experimental import pallas as pl
from jax.experimental.pallas import tpu as pltpu
from jax.experimental.pallas import tpu_sc as plsc

B = 1024
N = 100000
LAMBDA = 0.1
TN = 2048
GRID = 25


@functools.lru_cache(maxsize=1)
def _make_sc_gather():
    info = plsc.get_sparse_core_info()
    nc, ns = info.num_cores, info.num_subcores
    nw = nc * ns
    b_per_w = B // nw

    mesh = plsc.VectorSubcoreMesh(core_axis_name="c", subcore_axis_name="s")

    @functools.partial(
        pl.kernel,
        mesh=mesh,
        out_type=[
            jax.ShapeDtypeStruct((B,), jnp.int32),
            jax.ShapeDtypeStruct((B,), jnp.int32),
        ],
        scratch_types=[
            pltpu.VMEM((b_per_w,), jnp.int32),
            pltpu.VMEM((b_per_w,), jnp.int32),
            pltpu.VMEM((b_per_w,), jnp.int32),
            pltpu.SemaphoreType.DMA,
            pltpu.SemaphoreType.DMA,
        ],
    )
    def sc_gather(idx_hbm, cls_hbm, ins_hbm, clsy_hbm, insy_hbm,
                  idx_v, a_v, b_v, sem_a, sem_b):
        wid = lax.axis_index("s") * nc + lax.axis_index("c")
        base = wid * b_per_w
        pltpu.sync_copy(idx_hbm.at[pl.ds(base, b_per_w)], idx_v)
        cp_a = pltpu.async_copy(cls_hbm.at[idx_v], a_v, sem_a)
        cp_b = pltpu.async_copy(ins_hbm.at[idx_v], b_v, sem_b)
        cp_a.wait()
        cp_b.wait()
        pltpu.sync_copy(a_v, clsy_hbm.at[pl.ds(base, b_per_w)])
        pltpu.sync_copy(b_v, insy_hbm.at[pl.ds(base, b_per_w)])

    return sc_gather


def _masked_sums(x_ref, cls_ref, ins_ref, clsy_ref, insy_ref, idx_ref, tile):
    e = jnp.exp(x_ref[...])
    col = lax.broadcasted_iota(jnp.int32, (B, TN), 1) + tile * TN
    valid = (col < N) & (col != idx_ref[...])
    e = jnp.where(valid, e, 0.0)
    zp = jnp.sum(e, axis=1, keepdims=True)
    p1p = jnp.sum(jnp.where(cls_ref[...] == clsy_ref[...], e, 0.0),
                  axis=1, keepdims=True)
    p2p = jnp.sum(jnp.where(ins_ref[...] == insy_ref[...], e, 0.0),
                  axis=1, keepdims=True)
    return zp, p1p, p2p


def _tc_body(xa_ref, xb_ref, clsa_ref, clsb_ref, insa_ref, insb_ref,
             clsy_ref, insy_ref, idx_ref,
             out1_ref, out2_ref, zacc, p1acc, p2acc):
    k = pl.program_id(0)
    za, p1a, p2a = _masked_sums(xa_ref, clsa_ref, insa_ref,
                                clsy_ref, insy_ref, idx_ref, k)
    zb, p1b, p2b = _masked_sums(xb_ref, clsb_ref, insb_ref,
                                clsy_ref, insy_ref, idx_ref, k + GRID)
    zp = za + zb
    p1p = p1a + p1b
    p2p = p2a + p2b

    @pl.when(k == 0)
    def _init():
        zacc[...] = zp
        p1acc[...] = p1p
        p2acc[...] = p2p

    @pl.when(k > 0)
    def _accum():
        zacc[...] += zp
        p1acc[...] += p1p
        p2acc[...] += p2p

    @pl.when(k == GRID - 1)
    def _finalize():
        z = zacc[...]
        prob1 = p1acc[...] / z
        prob2 = p2acc[...] / z
        nz1 = prob1 != 0.0
        l1 = jnp.where(nz1, jnp.log(jnp.where(nz1, prob1, 1.0)), 0.0)
        nz2 = prob2 != 0.0
        l2 = jnp.where(nz2, jnp.log(jnp.where(nz2, prob2, 1.0)), 0.0)
        out1_ref[...] = (-jnp.sum(l1) / B).reshape(1, 1)
        out2_ref[...] = (-LAMBDA * jnp.sum(l2) / B).reshape(1, 1)


def _tc_call(x, cls2d, ins2d, clsy, insy, idx2d, interpret=False):
    out1, out2 = pl.pallas_call(
        _tc_body,
        grid=(GRID,),
        in_specs=[
            pl.BlockSpec((B, TN), lambda k: (k * 0, k)),
            pl.BlockSpec((B, TN), lambda k: (k * 0, jnp.minimum(k + GRID, 48))),
            pl.BlockSpec((1, TN), lambda k: (k * 0, k)),
            pl.BlockSpec((1, TN), lambda k: (k * 0, jnp.minimum(k + GRID, 48))),
            pl.BlockSpec((1, TN), lambda k: (k * 0, k)),
            pl.BlockSpec((1, TN), lambda k: (k * 0, jnp.minimum(k + GRID, 48))),
            pl.BlockSpec((B, 1), lambda k: (k * 0, k * 0)),
            pl.BlockSpec((B, 1), lambda k: (k * 0, k * 0)),
            pl.BlockSpec((B, 1), lambda k: (k * 0, k * 0)),
        ],
        out_specs=[
            pl.BlockSpec((1, 1), lambda k: (k * 0, k * 0)),
            pl.BlockSpec((1, 1), lambda k: (k * 0, k * 0)),
        ],
        out_shape=[
            jax.ShapeDtypeStruct((1, 1), jnp.float32),
            jax.ShapeDtypeStruct((1, 1), jnp.float32),
        ],
        scratch_shapes=[
            pltpu.VMEM((B, 1), jnp.float32),
            pltpu.VMEM((B, 1), jnp.float32),
            pltpu.VMEM((B, 1), jnp.float32),
        ],
        compiler_params=pltpu.CompilerParams(
            dimension_semantics=("arbitrary",),
        ),
        interpret=interpret,
    )(x, x, cls2d, cls2d, ins2d, ins2d, clsy, insy, idx2d)
    return out1, out2


def kernel(x, indexes, clsLabels, insLabels):
    idx32 = indexes.astype(jnp.int32)
    cls32 = clsLabels.astype(jnp.int32)
    ins32 = insLabels.astype(jnp.int32)
    clsy, insy = _make_sc_gather()(idx32, cls32, ins32)
    out1, out2 = _tc_call(
        x,
        cls32.reshape(1, N),
        ins32.reshape(1, N),
        clsy.reshape(B, 1),
        insy.reshape(B, 1),
        idx32.reshape(B, 1),
    )
    return (out1[0, 0], out2[0, 0])
